# baseline (speedup 1.0000x reference)
import functools

import jax
import jax.numpy as jnp
from jax import lax
from jax.experimental import pallas as pl
from jax.experimental.pallas import tpu as pltpu
from jax.experimental.pallas import tpu_sc as plsc

NN = 10000
EE = 320000
GG = 64
D = 128
P1 = 128
EPS = 1e-5

NC, NS = 2, 16
NW = NC * NS
EPT = EE // NW
CH = 64
NCHUNKP = 160
CBI = 16
NBUF = 3
PCH = 80
CPT = EE // NS
CNT = 3 * NN
CNTP = 30720
NP = 10240
ROWS_PT = NP // NS
RB = 128

_MESH = plsc.VectorSubcoreMesh(core_axis_name="c", subcore_axis_name="s")
_SC_PARAMS = pltpu.CompilerParams(needs_layout_passes=False)


def _pre_body(src_h, dst_h, et_h, g_h, coef_h,
              bufA, bufB, g2v, cntv, gv, coefv, onesv, zv, cnt_sh):
    cid = lax.axis_index("c")
    sid = lax.axis_index("s")
    wid = cid * NS + sid

    @pl.loop(0, 1920 // 16)
    def _(k):
        zv[pl.ds(k * 16, 16)] = jnp.zeros((16,), jnp.float32)
    pltpu.sync_copy(zv, cnt_sh.at[pl.ds(sid * 1920, 1920)])

    @pl.loop(0, PCH // 16)
    def _(k):
        onesv[pl.ds(k * 16, 16)] = jnp.ones((16,), jnp.float32)

    cbase = sid * CPT
    pltpu.sync_copy(dst_h.at[pl.ds(cbase, CPT)], bufA)
    pltpu.sync_copy(et_h.at[pl.ds(cbase, CPT)], bufB)

    @pl.loop(0, CPT // 16)
    def _(i):
        d16 = bufA[pl.ds(i * 16, 16)]
        e16 = bufB[pl.ds(i * 16, 16)]
        g2v[i // 5, pl.ds((i % 5) * 16, 16)] = e16 * NN + d16

    plsc.subcore_barrier()

    @pl.loop(0, CPT // PCH)
    def _(j):
        pltpu.sync_copy(onesv, cnt_sh.at[g2v.at[j]], add=True)

    plsc.subcore_barrier()
    pltpu.sync_copy(cnt_sh, cntv)

    ebase = wid * EPT
    pltpu.sync_copy(src_h.at[pl.ds(ebase, EPT)], bufA.at[pl.ds(0, EPT)])
    pltpu.sync_copy(dst_h.at[pl.ds(ebase, EPT)], bufA.at[pl.ds(EPT, EPT)])
    pltpu.sync_copy(et_h.at[pl.ds(ebase, EPT)], bufB.at[pl.ds(0, EPT)])

    @pl.loop(0, EPT // 16)
    def _(i):
        s16 = bufA[pl.ds(i * 16, 16)]
        d16 = bufA[pl.ds(EPT + i * 16, 16)]
        e16 = bufB[pl.ds(i * 16, 16)]
        gv[pl.ds(i * 16, 16)] = e16 * NN + s16
        c16 = plsc.load_gather(cntv, [e16 * NN + d16])
        coefv[pl.ds(i * 16, 16)] = 1.0 / jnp.maximum(c16, 1.0)

    pltpu.sync_copy(gv, g_h.at[wid])
    pltpu.sync_copy(coefv, coef_h.at[wid])


_pre = functools.partial(
    pl.kernel,
    out_type=(jax.ShapeDtypeStruct((NW, EPT), jnp.int32),
              jax.ShapeDtypeStruct((NW, EPT), jnp.float32)),
    mesh=_MESH,
    scratch_types=[
        pltpu.VMEM((CPT,), jnp.int32),
        pltpu.VMEM((CPT,), jnp.int32),
        pltpu.VMEM((CPT // PCH, PCH), jnp.int32),
        pltpu.VMEM((CNTP,), jnp.float32),
        pltpu.VMEM((EPT,), jnp.int32),
        pltpu.VMEM((EPT,), jnp.float32),
        pltpu.VMEM((PCH,), jnp.float32),
        pltpu.VMEM((1920,), jnp.float32),
        pltpu.VMEM_SHARED((CNTP,), jnp.float32),
    ],
    compiler_params=_SC_PARAMS,
)(_pre_body)


def _msg_body(hcat_h, g3_h, dst3_h, coef3_h, out_h,
              g_v, dst_v, coef_v, rows, sems, acc_sh):
    cid = lax.axis_index("c")
    sid = lax.axis_index("s")
    wid = cid * NS + sid

    @pl.loop(0, CH)
    def _(r):
        for k in range(D // 16):
            rows[0, r, pl.ds(k * 16, 16)] = jnp.zeros((16,), jnp.float32)

    @pl.loop(0, ROWS_PT // CH)
    def _(k):
        pltpu.sync_copy(rows.at[0], acc_sh.at[pl.ds(sid * ROWS_PT + k * CH, CH)])

    plsc.subcore_barrier()

    pltpu.sync_copy(g3_h.at[wid, pl.ds(0, CBI)], g_v)
    pltpu.sync_copy(dst3_h.at[wid, pl.ds(0, CBI)], dst_v)
    pltpu.sync_copy(coef3_h.at[wid, pl.ds(0, CBI)], coef_v)
    pltpu.async_copy(hcat_h.at[g_v.at[0]], rows.at[0], sems.at[0])
    pltpu.async_copy(hcat_h.at[g_v.at[1]], rows.at[1], sems.at[1])

    @pl.loop(0, NCHUNKP)
    def _(j):
        cb = lax.rem(j, CBI)
        b = lax.rem(j, NBUF)
        b2 = lax.rem(j + 2, NBUF)

        pltpu.make_async_copy(
            hcat_h.at[g_v.at[cb]], rows.at[b], sems.at[b]).wait()

        @pl.when(cb + 2 < CBI)
        def _():
            pltpu.async_copy(
                hcat_h.at[g_v.at[cb + 2]], rows.at[b2], sems.at[b2])

        @pl.loop(0, CH, unroll=2)
        def _(e):
            c16 = plsc.load_gather(
                coef_v, [jnp.full((16,), cb, jnp.int32),
                         jnp.full((16,), e, jnp.int32)])
            for k in range(D // 16):
                rows[b, e, pl.ds(k * 16, 16)] = (
                    rows[b, e, pl.ds(k * 16, 16)] * c16)

        pltpu.sync_copy(rows.at[b], acc_sh.at[dst_v.at[cb]], add=True)

        @pl.when(jnp.logical_and(cb == CBI - 1, j < NCHUNKP - 1))
        def _():
            blk = (j + 1) // CBI
            pltpu.sync_copy(g3_h.at[wid, pl.ds(blk * CBI, CBI)], g_v)
            pltpu.sync_copy(dst3_h.at[wid, pl.ds(blk * CBI, CBI)], dst_v)
            pltpu.sync_copy(coef3_h.at[wid, pl.ds(blk * CBI, CBI)], coef_v)
            b1 = lax.rem(j + 1, NBUF)
            pltpu.async_copy(hcat_h.at[g_v.at[0]], rows.at[b1], sems.at[b1])
            pltpu.async_copy(hcat_h.at[g_v.at[1]], rows.at[b2], sems.at[b2])

    plsc.subcore_barrier()

    @pl.loop(0, ROWS_PT // RB)
    def _(k):
        r0 = sid * ROWS_PT + k * RB
        pltpu.sync_copy(acc_sh.at[pl.ds(r0, RB)], out_h.at[cid, pl.ds(r0, RB)])


_msg = functools.partial(
    pl.kernel,
    out_type=jax.ShapeDtypeStruct((NC, NP, D), jnp.float32),
    mesh=_MESH,
    scratch_types=[
        pltpu.VMEM((CBI, CH), jnp.int32),
        pltpu.VMEM((CBI, CH), jnp.int32),
        pltpu.VMEM((CBI, CH), jnp.float32),
        pltpu.VMEM((NBUF, CH, D), jnp.float32),
        pltpu.SemaphoreType.DMA((NBUF,)),
        pltpu.VMEM_SHARED((NP, D), jnp.float32),
    ],
    compiler_params=_SC_PARAMS,
)(_msg_body)


RT = 400
NRT = NN // RT


def _dense_body(stats_ref, gamma_ref, beta_ref, h_ref, wcat_ref, root_ref,
                bias_ref, hcat_ref, rootp_ref):
    mu = stats_ref[0:1, :]
    var = stats_ref[1:2, :]
    a = gamma_ref[...] * lax.rsqrt(var + EPS)
    c = beta_ref[...] - mu * a
    hn = h_ref[...] * a + c
    for r in range(3):
        hcat_ref[r] = jnp.dot(hn, wcat_ref[r], preferred_element_type=jnp.float32)
    rootp_ref[...] = (jnp.dot(hn, root_ref[...], preferred_element_type=jnp.float32)
                      + bias_ref[...])


_dense = pl.pallas_call(
    _dense_body,
    grid=(NRT,),
    in_specs=[
        pl.BlockSpec((2, D), lambda i: (0, 0)),
        pl.BlockSpec((1, D), lambda i: (0, 0)),
        pl.BlockSpec((1, D), lambda i: (0, 0)),
        pl.BlockSpec((RT, D), lambda i: (i, 0)),
        pl.BlockSpec((3, D, D), lambda i: (0, 0, 0)),
        pl.BlockSpec((D, D), lambda i: (0, 0)),
        pl.BlockSpec((1, D), lambda i: (0, 0)),
    ],
    out_specs=[
        pl.BlockSpec((3, RT, D), lambda i: (0, i, 0)),
        pl.BlockSpec((RT, D), lambda i: (i, 0)),
    ],
    out_shape=[
        jax.ShapeDtypeStruct((3, NN, D), jnp.float32),
        jax.ShapeDtypeStruct((NN, D), jnp.float32),
    ],
)


def _stats_body(rootp_ref, part_ref, out_ref, stats_ref):
    i = pl.program_id(0)
    s = rootp_ref[...] + part_ref[0] + part_ref[1]
    out_ref[...] = s

    @pl.when(i == 0)
    def _():
        stats_ref[...] = jnp.zeros_like(stats_ref)

    stats_ref[0:1, :] += jnp.sum(s, axis=0, keepdims=True)
    stats_ref[1:2, :] += jnp.sum(s * s, axis=0, keepdims=True)


_stats = pl.pallas_call(
    _stats_body,
    grid=(NRT,),
    in_specs=[
        pl.BlockSpec((RT, D), lambda i: (i, 0)),
        pl.BlockSpec((2, RT, D), lambda i: (0, i, 0)),
    ],
    out_specs=[
        pl.BlockSpec((RT, D), lambda i: (i, 0)),
        pl.BlockSpec((2, D), lambda i: (0, 0)),
    ],
    out_shape=[
        jax.ShapeDtypeStruct((NN, D), jnp.float32),
        jax.ShapeDtypeStruct((2, D), jnp.float32),
    ],
)


def _pool_body(batch_ref, h_ref, stats_ref, gamma_ref, beta_ref, meta_ref,
               l1x_ref, l1m_ref, l1b_ref, l2_ref, l2b_ref, z_ref,
               pooled_acc, cnt_acc):
    i = pl.program_id(0)

    @pl.when(i == 0)
    def _():
        pooled_acc[...] = jnp.zeros_like(pooled_acc)
        cnt_acc[...] = jnp.zeros_like(cnt_acc)

    ids = batch_ref[0]
    rows_iota = lax.broadcasted_iota(jnp.int32, (GG, RT), 0)
    msk = (rows_iota == ids).astype(jnp.float32)
    pooled_acc[...] += jnp.dot(msk, h_ref[...], preferred_element_type=jnp.float32)
    cnt_acc[...] += jnp.sum(msk, axis=1, keepdims=True)

    @pl.when(i == NRT - 1)
    def _():
        mu = stats_ref[0:1, :] / NN
        var = stats_ref[1:2, :] / NN - mu * mu
        a = gamma_ref[...] * lax.rsqrt(var + EPS)
        c = beta_ref[...] - mu * a
        cnt = cnt_acc[:, 0:1]
        pm = pooled_acc[...] / jnp.maximum(cnt, 1.0)
        pn = pm * a + c
        z1 = (jnp.dot(pn, l1x_ref[...], preferred_element_type=jnp.float32)
              + jnp.dot(meta_ref[...], l1m_ref[...], preferred_element_type=jnp.float32)
              + l1b_ref[...])
        z_ref[...] = (jnp.dot(z1, l2_ref[...], preferred_element_type=jnp.float32)
                      + l2b_ref[...])


_pool = pl.pallas_call(
    _pool_body,
    grid=(NRT,),
    in_specs=[
        pl.BlockSpec((1, 1, RT), lambda i: (i, 0, 0)),
        pl.BlockSpec((RT, D), lambda i: (i, 0)),
        pl.BlockSpec((2, D), lambda i: (0, 0)),
        pl.BlockSpec((1, D), lambda i: (0, 0)),
        pl.BlockSpec((1, D), lambda i: (0, 0)),
        pl.BlockSpec((GG, 40), lambda i: (0, 0)),
        pl.BlockSpec((D, P1), lambda i: (0, 0)),
        pl.BlockSpec((40, P1), lambda i: (0, 0)),
        pl.BlockSpec((1, P1), lambda i: (0, 0)),
        pl.BlockSpec((P1, 8), lambda i: (0, 0)),
        pl.BlockSpec((1, 8), lambda i: (0, 0)),
    ],
    out_specs=pl.BlockSpec((GG, 8), lambda i: (0, 0)),
    out_shape=jax.ShapeDtypeStruct((GG, 8), jnp.float32),
    scratch_shapes=[
        pltpu.VMEM((GG, D), jnp.float32),
        pltpu.VMEM((GG, D), jnp.float32),
    ],
)


def _pad2(m, r, c):
    return jnp.pad(m, ((0, r - m.shape[0]), (0, c - m.shape[1])))


def kernel(x, edge_attr, edge_index, edge_type, MetaData, batch,
           w0, root0, b0, gamma0, beta0,
           w1, root1, b1, gamma1, beta1,
           w2, root2, b2, gamma2, beta2,
           w3, root3, b3, gamma3, beta3,
           lin1_w, lin1_b, lin2_w, lin2_b):
    f32 = jnp.float32
    src = edge_index[0]
    dst = edge_index[1]

    g2d, coef2d = _pre(src, dst, edge_type)
    padt = ((0, 0), (0, NCHUNKP * CH - EPT))
    g3 = jnp.pad(g2d, padt).reshape(NW, NCHUNKP, CH)
    dst3 = jnp.pad(dst.reshape(NW, EPT), padt).reshape(NW, NCHUNKP, CH)
    coef3 = jnp.pad(coef2d, padt).reshape(NW, NCHUNKP, CH)

    layers = []
    for (w, root, b, g, be) in ((w0, root0, b0, gamma0, beta0),
                                (w1, root1, b1, gamma1, beta1),
                                (w2, root2, b2, gamma2, beta2),
                                (w3, root3, b3, gamma3, beta3)):
        wcat = jnp.stack([_pad2(w[r], D, D) for r in range(3)])
        rootp = _pad2(root, D, D)
        biasp = jnp.pad(b, (0, D - b.shape[0])).reshape(1, D)
        gp = jnp.pad(g, (0, D - g.shape[0])).reshape(1, D)
        bp = jnp.pad(be, (0, D - be.shape[0])).reshape(1, D)
        layers.append((wcat, rootp, biasp, gp, bp))

    stats = jnp.concatenate(
        [jnp.zeros((1, D), f32), jnp.full((1, D), 1.0 - EPS, f32)], axis=0)
    gamma_prev = jnp.ones((1, D), f32)
    beta_prev = jnp.zeros((1, D), f32)

    h = x
    for l, (wcat, rootp, biasp, gp, bp) in enumerate(layers):
        hcat, rp = _dense(stats, gamma_prev, beta_prev, h, wcat, rootp, biasp)
        part = _msg(hcat.reshape(3 * NN, D), g3, dst3, coef3)
        h, stats = _stats(rp, part)
        gamma_prev, beta_prev = gp, bp

    batch3 = batch.reshape(NRT, 1, RT)
    metap = _pad2(MetaData, GG, 40)
    l1x = _pad2(lin1_w[:128], D, P1)
    l1m = _pad2(lin1_w[128:], 40, P1)
    l1b = jnp.pad(lin1_b, (0, P1 - lin1_b.shape[0])).reshape(1, P1)
    l2 = _pad2(lin2_w, P1, 8)
    l2b = jnp.pad(lin2_b, (0, 8 - lin2_b.shape[0])).reshape(1, 8)

    z = _pool(batch3, h, stats, gamma_prev, beta_prev, metap,
              l1x, l1m, l1b, l2, l2b)
    return z[:, :1]


# speedup vs baseline: 1.6192x; 1.6192x over previous
import functools

import jax
import jax.numpy as jnp
from jax import lax
from jax.experimental import pallas as pl
from jax.experimental.pallas import tpu as pltpu
from jax.experimental.pallas import tpu_sc as plsc

NN = 10000
EE = 320000
GG = 64
D = 128
P1 = 128
EPS = 1e-5

NC, NS = 2, 16
NW = NC * NS
EPT = EE // NW
CH = 80
NCHUNKP = 128
CBI = 16
PCH = 80
CPT = EE // NS
CNT = 3 * NN
CNTP = 30720
NP = 10240
ROWS_PT = NP // NS
RB = 128

_MESH = plsc.VectorSubcoreMesh(core_axis_name="c", subcore_axis_name="s")
_SC_PARAMS = pltpu.CompilerParams(needs_layout_passes=False)


def _pre_body(src_h, dst_h, et_h, g_h, coef_h,
              bufA, bufB, g2v, cntv, gv, coefv, onesv, zv, cnt_sh):
    cid = lax.axis_index("c")
    sid = lax.axis_index("s")
    wid = cid * NS + sid

    @pl.loop(0, 1920 // 16)
    def _(k):
        zv[pl.ds(k * 16, 16)] = jnp.zeros((16,), jnp.float32)
    pltpu.sync_copy(zv, cnt_sh.at[pl.ds(sid * 1920, 1920)])

    @pl.loop(0, PCH // 16)
    def _(k):
        onesv[pl.ds(k * 16, 16)] = jnp.ones((16,), jnp.float32)

    cbase = sid * CPT
    pltpu.sync_copy(dst_h.at[pl.ds(cbase, CPT)], bufA)
    pltpu.sync_copy(et_h.at[pl.ds(cbase, CPT)], bufB)

    @pl.loop(0, CPT // 16)
    def _(i):
        d16 = bufA[pl.ds(i * 16, 16)]
        e16 = bufB[pl.ds(i * 16, 16)]
        g2v[i // 5, pl.ds((i % 5) * 16, 16)] = e16 * NN + d16

    plsc.subcore_barrier()

    @pl.loop(0, CPT // PCH)
    def _(j):
        pltpu.sync_copy(onesv, cnt_sh.at[g2v.at[j]], add=True)

    plsc.subcore_barrier()
    pltpu.sync_copy(cnt_sh, cntv)

    ebase = wid * EPT
    pltpu.sync_copy(src_h.at[pl.ds(ebase, EPT)], bufA.at[pl.ds(0, EPT)])
    pltpu.sync_copy(dst_h.at[pl.ds(ebase, EPT)], bufA.at[pl.ds(EPT, EPT)])
    pltpu.sync_copy(et_h.at[pl.ds(ebase, EPT)], bufB.at[pl.ds(0, EPT)])

    @pl.loop(0, EPT // 16)
    def _(i):
        s16 = bufA[pl.ds(i * 16, 16)]
        d16 = bufA[pl.ds(EPT + i * 16, 16)]
        e16 = bufB[pl.ds(i * 16, 16)]
        gv[pl.ds(i * 16, 16)] = e16 * NN + s16
        c16 = plsc.load_gather(cntv, [e16 * NN + d16])
        coefv[pl.ds(i * 16, 16)] = 1.0 / jnp.maximum(c16, 1.0)

    pltpu.sync_copy(gv, g_h.at[wid])
    pltpu.sync_copy(coefv, coef_h.at[wid])


_pre = functools.partial(
    pl.kernel,
    out_type=(jax.ShapeDtypeStruct((NW, EPT), jnp.int32),
              jax.ShapeDtypeStruct((NW, EPT), jnp.float32)),
    mesh=_MESH,
    scratch_types=[
        pltpu.VMEM((CPT,), jnp.int32),
        pltpu.VMEM((CPT,), jnp.int32),
        pltpu.VMEM((CPT // PCH, PCH), jnp.int32),
        pltpu.VMEM((CNTP,), jnp.float32),
        pltpu.VMEM((EPT,), jnp.int32),
        pltpu.VMEM((EPT,), jnp.float32),
        pltpu.VMEM((PCH,), jnp.float32),
        pltpu.VMEM((1920,), jnp.float32),
        pltpu.VMEM_SHARED((CNTP,), jnp.float32),
    ],
    compiler_params=_SC_PARAMS,
)(_pre_body)


def _msg_body(hcat_h, g3_h, dst3_h, coef3_h, out_h,
              g_v, dst_v, coef_v, rows, sems, acc_sh):
    cid = lax.axis_index("c")
    sid = lax.axis_index("s")
    wid = cid * NS + sid

    @pl.loop(0, CH)
    def _(r):
        for k in range(D // 16):
            rows[0, r, pl.ds(k * 16, 16)] = jnp.zeros((16,), jnp.float32)

    @pl.loop(0, ROWS_PT // CH)
    def _(k):
        pltpu.sync_copy(rows.at[0], acc_sh.at[pl.ds(sid * ROWS_PT + k * CH, CH)])

    plsc.subcore_barrier()

    pltpu.sync_copy(g3_h.at[wid, pl.ds(0, CBI)], g_v)
    pltpu.sync_copy(dst3_h.at[wid, pl.ds(0, CBI)], dst_v)
    pltpu.sync_copy(coef3_h.at[wid, pl.ds(0, CBI)], coef_v)
    pltpu.async_copy(hcat_h.at[g_v.at[0]], rows.at[0], sems.at[0])

    @pl.loop(0, NCHUNKP // 2)
    def _(p):
        for b in range(2):
            j = p * 2 + b
            cb = lax.rem(j, CBI)

            pltpu.make_async_copy(
                hcat_h.at[g_v.at[cb]], rows.at[b], sems.at[b]).wait()

            @pl.when(jnp.logical_and(cb < CBI - 1, j < NCHUNKP - 1))
            def _():
                pltpu.async_copy(
                    hcat_h.at[g_v.at[cb + 1]], rows.at[1 - b], sems.at[1 - b])

            @pl.loop(0, CH, unroll=2)
            def _(e):
                c16 = plsc.load_gather(
                    coef_v, [jnp.full((16,), cb, jnp.int32),
                             jnp.full((16,), e, jnp.int32)])
                for k in range(D // 16):
                    rows[b, e, pl.ds(k * 16, 16)] = (
                        rows[b, e, pl.ds(k * 16, 16)] * c16)

            pltpu.sync_copy(rows.at[b], acc_sh.at[dst_v.at[cb]], add=True)

            @pl.when(jnp.logical_and(cb == CBI - 1, j < NCHUNKP - 1))
            def _():
                blk = (j + 1) // CBI
                pltpu.sync_copy(g3_h.at[wid, pl.ds(blk * CBI, CBI)], g_v)
                pltpu.sync_copy(dst3_h.at[wid, pl.ds(blk * CBI, CBI)], dst_v)
                pltpu.sync_copy(coef3_h.at[wid, pl.ds(blk * CBI, CBI)], coef_v)
                pltpu.async_copy(
                    hcat_h.at[g_v.at[0]], rows.at[1 - b], sems.at[1 - b])

    plsc.subcore_barrier()

    @pl.loop(0, ROWS_PT // RB)
    def _(k):
        r0 = sid * ROWS_PT + k * RB
        pltpu.sync_copy(acc_sh.at[pl.ds(r0, RB)], out_h.at[cid, pl.ds(r0, RB)])


_msg = functools.partial(
    pl.kernel,
    out_type=jax.ShapeDtypeStruct((NC, NP, D), jnp.float32),
    mesh=_MESH,
    scratch_types=[
        pltpu.VMEM((CBI, CH), jnp.int32),
        pltpu.VMEM((CBI, CH), jnp.int32),
        pltpu.VMEM((CBI, CH), jnp.float32),
        pltpu.VMEM((2, CH, D), jnp.float32),
        pltpu.SemaphoreType.DMA((2,)),
        pltpu.VMEM_SHARED((NP, D), jnp.float32),
    ],
    compiler_params=_SC_PARAMS,
)(_msg_body)


RT = 400
NRT = NN // RT


def _dense_body(stats_ref, gamma_ref, beta_ref, h_ref, wcat_ref, root_ref,
                bias_ref, hcat_ref, rootp_ref):
    mu = stats_ref[0:1, :]
    var = stats_ref[1:2, :]
    a = gamma_ref[...] * lax.rsqrt(var + EPS)
    c = beta_ref[...] - mu * a
    hn = h_ref[...] * a + c
    for r in range(3):
        hcat_ref[r] = jnp.dot(hn, wcat_ref[r], preferred_element_type=jnp.float32)
    rootp_ref[...] = (jnp.dot(hn, root_ref[...], preferred_element_type=jnp.float32)
                      + bias_ref[...])


_dense = pl.pallas_call(
    _dense_body,
    grid=(NRT,),
    in_specs=[
        pl.BlockSpec((2, D), lambda i: (0, 0)),
        pl.BlockSpec((1, D), lambda i: (0, 0)),
        pl.BlockSpec((1, D), lambda i: (0, 0)),
        pl.BlockSpec((RT, D), lambda i: (i, 0)),
        pl.BlockSpec((3, D, D), lambda i: (0, 0, 0)),
        pl.BlockSpec((D, D), lambda i: (0, 0)),
        pl.BlockSpec((1, D), lambda i: (0, 0)),
    ],
    out_specs=[
        pl.BlockSpec((3, RT, D), lambda i: (0, i, 0)),
        pl.BlockSpec((RT, D), lambda i: (i, 0)),
    ],
    out_shape=[
        jax.ShapeDtypeStruct((3, NN, D), jnp.float32),
        jax.ShapeDtypeStruct((NN, D), jnp.float32),
    ],
)


def _stats_body(rootp_ref, part_ref, out_ref, stats_ref):
    i = pl.program_id(0)
    s = rootp_ref[...] + part_ref[0] + part_ref[1]
    out_ref[...] = s

    @pl.when(i == 0)
    def _():
        stats_ref[...] = jnp.zeros_like(stats_ref)

    stats_ref[0:1, :] += jnp.sum(s, axis=0, keepdims=True)
    stats_ref[1:2, :] += jnp.sum(s * s, axis=0, keepdims=True)


_stats = pl.pallas_call(
    _stats_body,
    grid=(NRT,),
    in_specs=[
        pl.BlockSpec((RT, D), lambda i: (i, 0)),
        pl.BlockSpec((2, RT, D), lambda i: (0, i, 0)),
    ],
    out_specs=[
        pl.BlockSpec((RT, D), lambda i: (i, 0)),
        pl.BlockSpec((2, D), lambda i: (0, 0)),
    ],
    out_shape=[
        jax.ShapeDtypeStruct((NN, D), jnp.float32),
        jax.ShapeDtypeStruct((2, D), jnp.float32),
    ],
)


def _pool_body(batch_ref, h_ref, stats_ref, gamma_ref, beta_ref, meta_ref,
               l1x_ref, l1m_ref, l1b_ref, l2_ref, l2b_ref, z_ref,
               pooled_acc, cnt_acc):
    i = pl.program_id(0)

    @pl.when(i == 0)
    def _():
        pooled_acc[...] = jnp.zeros_like(pooled_acc)
        cnt_acc[...] = jnp.zeros_like(cnt_acc)

    ids = batch_ref[0]
    rows_iota = lax.broadcasted_iota(jnp.int32, (GG, RT), 0)
    msk = (rows_iota == ids).astype(jnp.float32)
    pooled_acc[...] += jnp.dot(msk, h_ref[...], preferred_element_type=jnp.float32)
    cnt_acc[...] += jnp.sum(msk, axis=1, keepdims=True)

    @pl.when(i == NRT - 1)
    def _():
        mu = stats_ref[0:1, :] / NN
        var = stats_ref[1:2, :] / NN - mu * mu
        a = gamma_ref[...] * lax.rsqrt(var + EPS)
        c = beta_ref[...] - mu * a
        cnt = cnt_acc[:, 0:1]
        pm = pooled_acc[...] / jnp.maximum(cnt, 1.0)
        pn = pm * a + c
        z1 = (jnp.dot(pn, l1x_ref[...], preferred_element_type=jnp.float32)
              + jnp.dot(meta_ref[...], l1m_ref[...], preferred_element_type=jnp.float32)
              + l1b_ref[...])
        z_ref[...] = (jnp.dot(z1, l2_ref[...], preferred_element_type=jnp.float32)
                      + l2b_ref[...])


_pool = pl.pallas_call(
    _pool_body,
    grid=(NRT,),
    in_specs=[
        pl.BlockSpec((1, 1, RT), lambda i: (i, 0, 0)),
        pl.BlockSpec((RT, D), lambda i: (i, 0)),
        pl.BlockSpec((2, D), lambda i: (0, 0)),
        pl.BlockSpec((1, D), lambda i: (0, 0)),
        pl.BlockSpec((1, D), lambda i: (0, 0)),
        pl.BlockSpec((GG, 40), lambda i: (0, 0)),
        pl.BlockSpec((D, P1), lambda i: (0, 0)),
        pl.BlockSpec((40, P1), lambda i: (0, 0)),
        pl.BlockSpec((1, P1), lambda i: (0, 0)),
        pl.BlockSpec((P1, 8), lambda i: (0, 0)),
        pl.BlockSpec((1, 8), lambda i: (0, 0)),
    ],
    out_specs=pl.BlockSpec((GG, 8), lambda i: (0, 0)),
    out_shape=jax.ShapeDtypeStruct((GG, 8), jnp.float32),
    scratch_shapes=[
        pltpu.VMEM((GG, D), jnp.float32),
        pltpu.VMEM((GG, D), jnp.float32),
    ],
)


def _pad2(m, r, c):
    return jnp.pad(m, ((0, r - m.shape[0]), (0, c - m.shape[1])))


def kernel(x, edge_attr, edge_index, edge_type, MetaData, batch,
           w0, root0, b0, gamma0, beta0,
           w1, root1, b1, gamma1, beta1,
           w2, root2, b2, gamma2, beta2,
           w3, root3, b3, gamma3, beta3,
           lin1_w, lin1_b, lin2_w, lin2_b):
    f32 = jnp.float32
    src = edge_index[0]
    dst = edge_index[1]

    g2d, coef2d = _pre(src, dst, edge_type)
    padt = ((0, 0), (0, NCHUNKP * CH - EPT))
    g3 = jnp.pad(g2d, padt).reshape(NW, NCHUNKP, CH)
    dst3 = jnp.pad(dst.reshape(NW, EPT), padt).reshape(NW, NCHUNKP, CH)
    coef3 = jnp.pad(coef2d, padt).reshape(NW, NCHUNKP, CH)

    layers = []
    for (w, root, b, g, be) in ((w0, root0, b0, gamma0, beta0),
                                (w1, root1, b1, gamma1, beta1),
                                (w2, root2, b2, gamma2, beta2),
                                (w3, root3, b3, gamma3, beta3)):
        wcat = jnp.stack([_pad2(w[r], D, D) for r in range(3)])
        rootp = _pad2(root, D, D)
        biasp = jnp.pad(b, (0, D - b.shape[0])).reshape(1, D)
        gp = jnp.pad(g, (0, D - g.shape[0])).reshape(1, D)
        bp = jnp.pad(be, (0, D - be.shape[0])).reshape(1, D)
        layers.append((wcat, rootp, biasp, gp, bp))

    stats = jnp.concatenate(
        [jnp.zeros((1, D), f32), jnp.full((1, D), 1.0 - EPS, f32)], axis=0)
    gamma_prev = jnp.ones((1, D), f32)
    beta_prev = jnp.zeros((1, D), f32)

    h = x
    for l, (wcat, rootp, biasp, gp, bp) in enumerate(layers):
        hcat, rp = _dense(stats, gamma_prev, beta_prev, h, wcat, rootp, biasp)
        part = _msg(hcat.reshape(3 * NN, D), g3, dst3, coef3)
        h, stats = _stats(rp, part)
        gamma_prev, beta_prev = gp, bp

    batch3 = batch.reshape(NRT, 1, RT)
    metap = _pad2(MetaData, GG, 40)
    l1x = _pad2(lin1_w[:128], D, P1)
    l1m = _pad2(lin1_w[128:], 40, P1)
    l1b = jnp.pad(lin1_b, (0, P1 - lin1_b.shape[0])).reshape(1, P1)
    l2 = _pad2(lin2_w, P1, 8)
    l2b = jnp.pad(lin2_b, (0, 8 - lin2_b.shape[0])).reshape(1, 8)

    z = _pool(batch3, h, stats, gamma_prev, beta_prev, metap,
              l1x, l1m, l1b, l2, l2b)
    return z[:, :1]
